# Initial kernel scaffold; baseline (speedup 1.0000x reference)
import jax
import jax.numpy as jnp
from jax.experimental import pallas as pl


def kernel(x, adj, sampled_neighbors, W1, b1, W2, b2):
    raise NotImplementedError("write your pallas kernel here")


# speedup vs baseline: 1.6263x; 1.6263x over previous
import functools

import jax
import jax.numpy as jnp
from jax import lax
from jax.experimental import pallas as pl
from jax.experimental.pallas import tpu as pltpu
from jax.experimental.pallas import tpu_sc as plsc

_NC, _NS = 2, 16
_NW = _NC * _NS
_D = 128
_S = 32
_NPAD = 10240
_NPW = _NPAD // _NW
_CN = 4
_NCHUNK = _NPW // _CN


def _gather_mean(table, idx_flat):
    mesh = plsc.VectorSubcoreMesh(core_axis_name="c", subcore_axis_name="s")

    @functools.partial(
        pl.kernel,
        out_type=jax.ShapeDtypeStruct((_NPAD, _D), jnp.float32),
        mesh=mesh,
        scratch_types=[
            pltpu.VMEM((_NPW * _S,), jnp.int32),
            pltpu.VMEM((_CN * _S, _D), jnp.float32),
            pltpu.VMEM((_NPW, _D), jnp.float32),
            pltpu.SemaphoreType.DMA,
        ],
    )
    def k(table_hbm, idx_hbm, out_hbm, idx_v, rows_v, out_v, sem):
        wid = lax.axis_index("s") * _NC + lax.axis_index("c")
        ibase = wid * (_NPW * _S)
        pltpu.sync_copy(idx_hbm.at[pl.ds(ibase, _NPW * _S)], idx_v)

        def chunk_body(c, carry):
            pltpu.async_copy(
                table_hbm.at[idx_v.at[pl.ds(c * (_CN * _S), _CN * _S)]],
                rows_v, sem).wait()
            for j in range(_CN):
                def row_body(r, accs):
                    row = j * _S + r
                    return tuple(accs[g] + rows_v[row, pl.ds(g * 16, 16)]
                                 for g in range(8))
                accs = lax.fori_loop(
                    0, _S, row_body,
                    tuple(jnp.zeros((16,), jnp.float32) for _ in range(8)))
                node = c * _CN + j
                for g in range(8):
                    out_v[node, pl.ds(g * 16, 16)] = accs[g] * (1.0 / _S)
            return carry

        lax.fori_loop(0, _NCHUNK, chunk_body, 0)
        pltpu.sync_copy(out_v, out_hbm.at[pl.ds(wid * _NPW, _NPW)])

    return k(table, idx_flat)


def _sage_linear(a, b, wa, wb, bias, relu):
    npad = a.shape[0]
    bm = 512

    def mm(a_ref, b_ref, wa_ref, wb_ref, bias_ref, o_ref):
        acc = jnp.dot(a_ref[...], wa_ref[...],
                      preferred_element_type=jnp.float32)
        acc = acc + jnp.dot(b_ref[...], wb_ref[...],
                            preferred_element_type=jnp.float32)
        acc = acc + bias_ref[...]
        if relu:
            acc = jnp.maximum(acc, 0.0)
        o_ref[...] = acc

    return pl.pallas_call(
        mm,
        grid=(npad // bm,),
        in_specs=[
            pl.BlockSpec((bm, _D), lambda i: (i, 0)),
            pl.BlockSpec((bm, _D), lambda i: (i, 0)),
            pl.BlockSpec((_D, _D), lambda i: (0, 0)),
            pl.BlockSpec((_D, _D), lambda i: (0, 0)),
            pl.BlockSpec((1, _D), lambda i: (0, 0)),
        ],
        out_specs=pl.BlockSpec((bm, _D), lambda i: (i, 0)),
        out_shape=jax.ShapeDtypeStruct((npad, _D), jnp.float32),
    )(a, b, wa, wb, bias)


def kernel(x, adj, sampled_neighbors, W1, b1, W2, b2):
    n, d = x.shape
    xp = jnp.zeros((_NPAD, d), x.dtype).at[:n].set(x)
    nbrp = jnp.concatenate(
        [sampled_neighbors,
         jnp.zeros((2, _NPAD - n, _S), sampled_neighbors.dtype)], axis=1)
    idx0 = nbrp[0].reshape(-1)
    idx1 = nbrp[1].reshape(-1)
    w1a, w1b = W1[:d], W1[d:]
    w2a, w2b = W2[:d], W2[d:]

    g1 = _gather_mean(xp, idx0)
    h1 = _sage_linear(xp, g1, w1a, w1b, b1.reshape(1, d), relu=True)
    g2 = _gather_mean(h1, idx1)
    h2 = _sage_linear(h1, g2, w2a, w2b, b2.reshape(1, d), relu=False)
    return h2[:n]


# speedup vs baseline: 1.8933x; 1.1642x over previous
import functools

import jax
import jax.numpy as jnp
from jax import lax
from jax.experimental import pallas as pl
from jax.experimental.pallas import tpu as pltpu
from jax.experimental.pallas import tpu_sc as plsc

_NC, _NS = 2, 16
_NW = _NC * _NS
_D = 128
_S = 32
_NPAD = 10240
_NPW = _NPAD // _NW
_CN = 4
_NCHUNK = _NPW // _CN


def _gather_mean(table, idx_flat):
    mesh = plsc.VectorSubcoreMesh(core_axis_name="c", subcore_axis_name="s")

    @functools.partial(
        pl.kernel,
        out_type=jax.ShapeDtypeStruct((_NPAD, _D), jnp.float32),
        mesh=mesh,
        scratch_types=[
            pltpu.VMEM((_NPW * _S,), jnp.int32),
            pltpu.VMEM((_CN * _S, _D), jnp.float32),
            pltpu.VMEM((_CN * _S, _D), jnp.float32),
            pltpu.VMEM((_NPW, _D), jnp.float32),
            pltpu.SemaphoreType.DMA,
            pltpu.SemaphoreType.DMA,
        ],
    )
    def k(table_hbm, idx_hbm, out_hbm, idx_v, rows0_v, rows1_v, out_v,
          sem0, sem1):
        wid = lax.axis_index("s") * _NC + lax.axis_index("c")
        ibase = wid * (_NPW * _S)
        pltpu.sync_copy(idx_hbm.at[pl.ds(ibase, _NPW * _S)], idx_v)

        def start(c, rows_v, sem):
            return pltpu.async_copy(
                table_hbm.at[idx_v.at[pl.ds(c * (_CN * _S), _CN * _S)]],
                rows_v, sem)

        def reduce_chunk(c, rows_v):
            for j in range(_CN):
                def row_body(r, accs):
                    row = j * _S + r * 4
                    for u in range(4):
                        accs = tuple(
                            accs[g] + rows_v[row + u, pl.ds(g * 16, 16)]
                            for g in range(8))
                    return accs
                accs = lax.fori_loop(
                    0, _S // 4, row_body,
                    tuple(jnp.zeros((16,), jnp.float32) for _ in range(8)))
                node = c * _CN + j
                for g in range(8):
                    out_v[node, pl.ds(g * 16, 16)] = accs[g] * (1.0 / _S)

        start(0, rows0_v, sem0)

        def pair_body(t, carry):
            a = t * 2
            start(a + 1, rows1_v, sem1)
            pltpu.make_async_copy(
                table_hbm.at[idx_v.at[pl.ds(0, _CN * _S)]],
                rows0_v, sem0).wait()
            reduce_chunk(a, rows0_v)
            start(jnp.minimum(a + 2, _NCHUNK - 1), rows0_v, sem0)
            pltpu.make_async_copy(
                table_hbm.at[idx_v.at[pl.ds(0, _CN * _S)]],
                rows1_v, sem1).wait()
            reduce_chunk(a + 1, rows1_v)
            return carry

        lax.fori_loop(0, _NCHUNK // 2, pair_body, 0)
        pltpu.make_async_copy(
            table_hbm.at[idx_v.at[pl.ds(0, _CN * _S)]],
            rows0_v, sem0).wait()
        pltpu.sync_copy(out_v, out_hbm.at[pl.ds(wid * _NPW, _NPW)])

    return k(table, idx_flat)


def _sage_linear(a, b, wa, wb, bias, relu):
    npad = a.shape[0]
    bm = 512

    def mm(a_ref, b_ref, wa_ref, wb_ref, bias_ref, o_ref):
        acc = jnp.dot(a_ref[...], wa_ref[...],
                      preferred_element_type=jnp.float32)
        acc = acc + jnp.dot(b_ref[...], wb_ref[...],
                            preferred_element_type=jnp.float32)
        acc = acc + bias_ref[...]
        if relu:
            acc = jnp.maximum(acc, 0.0)
        o_ref[...] = acc

    return pl.pallas_call(
        mm,
        grid=(npad // bm,),
        in_specs=[
            pl.BlockSpec((bm, _D), lambda i: (i, 0)),
            pl.BlockSpec((bm, _D), lambda i: (i, 0)),
            pl.BlockSpec((_D, _D), lambda i: (0, 0)),
            pl.BlockSpec((_D, _D), lambda i: (0, 0)),
            pl.BlockSpec((1, _D), lambda i: (0, 0)),
        ],
        out_specs=pl.BlockSpec((bm, _D), lambda i: (i, 0)),
        out_shape=jax.ShapeDtypeStruct((npad, _D), jnp.float32),
    )(a, b, wa, wb, bias)


def kernel(x, adj, sampled_neighbors, W1, b1, W2, b2):
    n, d = x.shape
    xp = jnp.zeros((_NPAD, d), x.dtype).at[:n].set(x)
    nbrp = jnp.concatenate(
        [sampled_neighbors,
         jnp.zeros((2, _NPAD - n, _S), sampled_neighbors.dtype)], axis=1)
    idx0 = nbrp[0].reshape(-1)
    idx1 = nbrp[1].reshape(-1)
    w1a, w1b = W1[:d], W1[d:]
    w2a, w2b = W2[:d], W2[d:]

    g1 = _gather_mean(xp, idx0)
    h1 = _sage_linear(xp, g1, w1a, w1b, b1.reshape(1, d), relu=True)
    g2 = _gather_mean(h1, idx1)
    h2 = _sage_linear(h1, g2, w2a, w2b, b2.reshape(1, d), relu=False)
    return h2[:n]
